# baseline (speedup 1.0000x reference)
import functools

import jax
import jax.numpy as jnp
from jax import lax
from jax.experimental import pallas as pl
from jax.experimental.pallas import tpu as pltpu
from jax.experimental.pallas import tpu_sc as plsc

VOCAB = 1000000
HIDDEN = 64
SCALE = 8.0
LANES = 16
NUM_CORES = 2
NUM_SUBCORES = 16
NW = NUM_CORES * NUM_SUBCORES
BATCH = 4096
SEQ = 200
BW = BATCH // NW
UNITS = VOCAB // 2
CBLK = 128
NBLK = VOCAB // CBLK
TAIL0 = NBLK * CBLK
TAILW = VOCAB - TAIL0
BSTEPS = -(-NBLK // NW)

_mesh = plsc.VectorSubcoreMesh(core_axis_name="c", subcore_axis_name="s")
_params = pltpu.CompilerParams(use_tc_tiling_on_sc=True, needs_layout_passes=False)


@functools.partial(
    pl.kernel,
    out_type=jax.ShapeDtypeStruct((UNITS, 2 * HIDDEN), jnp.float32),
    mesh=_mesh,
    scratch_types=[
        pltpu.VMEM((HIDDEN, CBLK), jnp.float32),
        pltpu.VMEM((CBLK // 2, 2 * HIDDEN), jnp.float32),
        pltpu.VMEM((HIDDEN, TAILW), jnp.float32),
        pltpu.VMEM((TAILW // 2, 2 * HIDDEN), jnp.float32),
    ],
    compiler_params=_params,
)
def _repack(wt_hbm, tab_hbm, in_v, out_v, tin_v, tout_v):
    wid = lax.axis_index("s") * NUM_CORES + lax.axis_index("c")
    ci = [lax.iota(jnp.int32, LANES) + g * LANES for g in range(CBLK // LANES)]
    rows = [lax.shift_right_logical(c, 1) for c in ci]
    cols = [jnp.bitwise_and(c, 1) * HIDDEN for c in ci]

    def step(t, carry):
        b = wid + t * NW

        @pl.when(b < NBLK)
        def _():
            c0 = b * CBLK
            pltpu.sync_copy(
                wt_hbm.at[pl.ds(0, HIDDEN), pl.ds(c0, CBLK)], in_v
            )
            for h in range(HIDDEN):
                for g in range(CBLK // LANES):
                    seg = in_v[h, pl.ds(g * LANES, LANES)]
                    plsc.store_scatter(out_v, [rows[g], cols[g] + h], seg)
            pltpu.sync_copy(out_v, tab_hbm.at[pl.ds(b * (CBLK // 2), CBLK // 2)])

        return carry

    lax.fori_loop(0, BSTEPS, step, 0)

    @pl.when(wid == NW - 1)
    def _tail():
        pltpu.sync_copy(wt_hbm.at[pl.ds(0, HIDDEN), pl.ds(TAIL0, TAILW)], tin_v)
        for h in range(HIDDEN):
            for g in range(TAILW // LANES):
                seg = tin_v[h, pl.ds(g * LANES, LANES)]
                plsc.store_scatter(tout_v, [rows[g], cols[g] + h], seg)
        pltpu.sync_copy(
            tout_v, tab_hbm.at[pl.ds(TAIL0 // 2, TAILW // 2)]
        )


@functools.partial(
    pl.kernel,
    out_type=jax.ShapeDtypeStruct((SEQ, HIDDEN, BATCH), jnp.float32),
    mesh=_mesh,
    scratch_types=[
        pltpu.VMEM((SEQ, BW), jnp.int32),
        pltpu.VMEM((BW,), jnp.int32),
        pltpu.VMEM((BW, 2 * HIDDEN), jnp.float32),
        pltpu.VMEM((HIDDEN, BW), jnp.float32),
        pltpu.SemaphoreType.DMA,
    ],
    compiler_params=_params,
)
def _embed(xt_hbm, tab_hbm, out_hbm, idx_v, uidx_v, unit_v, ot_v, sem):
    wid = lax.axis_index("s") * NUM_CORES + lax.axis_index("c")
    b0 = wid * BW

    pltpu.sync_copy(xt_hbm.at[pl.ds(0, SEQ), pl.ds(b0, BW)], idx_v)
    lane_iota = lax.iota(jnp.int32, LANES)

    def step(s, carry):
        for g in range(BW // LANES):
            iv = idx_v[s, pl.ds(g * LANES, LANES)]
            uidx_v[pl.ds(g * LANES, LANES)] = lax.shift_right_logical(iv, 1)
        pltpu.async_copy(tab_hbm.at[uidx_v], unit_v, sem).wait()

        def group(g, gc):
            iv = idx_v[s, pl.ds(g * LANES, LANES)]
            sv = jnp.where(iv != 0, jnp.float32(SCALE), jnp.float32(0.0))
            pv = jnp.bitwise_and(iv, 1) * HIDDEN
            for j in range(LANES):
                r = g * LANES + j
                scale = sv[j]
                off = pv[j]
                for c in range(HIDDEN // LANES):
                    seg = unit_v[r, pl.ds(off + c * LANES, LANES)]
                    plsc.store_scatter(
                        ot_v,
                        [c * LANES + lane_iota, jnp.full((LANES,), r, jnp.int32)],
                        seg * scale,
                    )
            return gc

        lax.fori_loop(0, BW // LANES, group, 0)
        pltpu.sync_copy(
            ot_v,
            out_hbm.at[s, pl.ds(0, HIDDEN), pl.ds(b0, BW)],
        )
        return carry

    lax.fori_loop(0, SEQ, step, 0)


def kernel(x, shared_weights):
    xt = x.astype(jnp.int32).T
    wt = shared_weights.T
    tab = _repack(wt)
    out_st = _embed(xt, tab)
    return jnp.transpose(out_st, (2, 0, 1))


# speedup vs baseline: 1.2375x; 1.2375x over previous
import functools

import jax
import jax.numpy as jnp
from jax import lax
from jax.experimental import pallas as pl
from jax.experimental.pallas import tpu as pltpu
from jax.experimental.pallas import tpu_sc as plsc

VOCAB = 1000000
HIDDEN = 64
SCALE = 8.0
LANES = 16
NUM_CORES = 2
NUM_SUBCORES = 16
NW = NUM_CORES * NUM_SUBCORES
BATCH = 4096
SEQ = 200
BW = BATCH // NW
TW = 2 * HIDDEN
CBLK = 128
NBLK = VOCAB // CBLK
TAIL0 = NBLK * CBLK
TAILW = VOCAB - TAIL0
RSTEPS = -(-NBLK // NW)
RBUF = 2
GBUF = 4
GOUT = SEQ // GBUF

_mesh = plsc.VectorSubcoreMesh(core_axis_name="c", subcore_axis_name="s")
_params = pltpu.CompilerParams(use_tc_tiling_on_sc=True, needs_layout_passes=False)


@functools.partial(
    pl.kernel,
    out_type=jax.ShapeDtypeStruct((VOCAB, TW), jnp.float32),
    mesh=_mesh,
    scratch_types=[
        pltpu.VMEM((RBUF, HIDDEN, CBLK), jnp.float32),
        pltpu.VMEM((RBUF, CBLK, TW), jnp.float32),
        pltpu.VMEM((HIDDEN, TAILW), jnp.float32),
        pltpu.VMEM((TAILW, TW), jnp.float32),
        pltpu.SemaphoreType.DMA,
        pltpu.SemaphoreType.DMA,
        pltpu.SemaphoreType.DMA,
        pltpu.SemaphoreType.DMA,
    ],
    compiler_params=_params,
)
def _repack(wt_hbm, tab_hbm, in_v, out_v, tin_v, tout_v, si0, si1, so0, so1):
    wid = lax.axis_index("s") * NUM_CORES + lax.axis_index("c")
    sin = [si0, si1]
    sout = [so0, so1]
    ci = [lax.iota(jnp.int32, LANES) + g * LANES for g in range(CBLK // LANES)]
    scale = jnp.float32(SCALE)

    def in_cp(t, k):
        c0 = (wid + t * NW) * CBLK
        return pltpu.make_async_copy(
            wt_hbm.at[pl.ds(0, HIDDEN), pl.ds(c0, CBLK)], in_v.at[k], sin[k]
        )

    def out_cp(t, k):
        c0 = (wid + t * NW) * CBLK
        return pltpu.make_async_copy(
            out_v.at[k], tab_hbm.at[pl.ds(c0, CBLK)], sout[k]
        )

    for k in range(RBUF):
        @pl.when(wid + k * NW < NBLK)
        def _():
            in_cp(k, k).start()

    def outer(T, carry):
        for k in range(RBUF):
            t = T * RBUF + k

            @pl.when(wid + t * NW < NBLK)
            def _():
                in_cp(t, k).wait()

                @pl.when(t >= RBUF)
                def _():
                    out_cp(t - RBUF, k).wait()

                for h in range(HIDDEN):
                    hv = jnp.full((LANES,), h, jnp.int32)
                    for g in range(CBLK // LANES):
                        seg = in_v[k, h, pl.ds(g * LANES, LANES)]
                        plsc.store_scatter(
                            out_v.at[k], [ci[g], hv], seg * scale
                        )
                out_cp(t, k).start()
                tn = t + RBUF

                @pl.when(wid + tn * NW < NBLK)
                def _():
                    in_cp(tn, k).start()

        return carry

    lax.fori_loop(0, -(-RSTEPS // RBUF), outer, 0)
    for k in range(RBUF):
        t_last = RSTEPS - RBUF + k

        @pl.when((t_last >= 0) & (wid + t_last * NW < NBLK))
        def _():
            out_cp(t_last, k).wait()

    @pl.when(wid == NW - 1)
    def _tail():
        pltpu.sync_copy(wt_hbm.at[pl.ds(0, HIDDEN), pl.ds(TAIL0, TAILW)], tin_v)
        for h in range(HIDDEN):
            hv = jnp.full((LANES,), h, jnp.int32)
            for g in range(TAILW // LANES):
                seg = tin_v[h, pl.ds(g * LANES, LANES)]
                plsc.store_scatter(tout_v, [ci[g], hv], seg * scale)
        pltpu.sync_copy(tout_v, tab_hbm.at[pl.ds(TAIL0, TAILW)])


@functools.partial(
    pl.kernel,
    out_type=jax.ShapeDtypeStruct((SEQ, HIDDEN, BATCH), jnp.float32),
    mesh=_mesh,
    scratch_types=[
        pltpu.VMEM((SEQ, BW), jnp.int32),
        pltpu.VMEM((GBUF, BW, TW), jnp.float32),
        pltpu.VMEM((GBUF, HIDDEN, BW), jnp.float32),
        pltpu.SemaphoreType.DMA,
        pltpu.SemaphoreType.DMA,
        pltpu.SemaphoreType.DMA,
        pltpu.SemaphoreType.DMA,
        pltpu.SemaphoreType.DMA,
        pltpu.SemaphoreType.DMA,
        pltpu.SemaphoreType.DMA,
        pltpu.SemaphoreType.DMA,
    ],
    compiler_params=_params,
)
def _embed(
    xt_hbm, tab_hbm, out_hbm, idx_v, unit_v, ot_v,
    g0, g1, g2, g3, o0, o1, o2, o3,
):
    wid = lax.axis_index("s") * NUM_CORES + lax.axis_index("c")
    b0 = wid * BW
    gsem = [g0, g1, g2, g3]
    osem = [o0, o1, o2, o3]
    lane_iota = lax.iota(jnp.int32, LANES)
    zeros = jnp.zeros((LANES,), jnp.float32)

    pltpu.sync_copy(xt_hbm.at[pl.ds(0, SEQ), pl.ds(b0, BW)], idx_v)

    def g_cp(s, k):
        return pltpu.make_async_copy(
            tab_hbm.at[idx_v.at[s]], unit_v.at[k], gsem[k]
        )

    def o_cp(s, k):
        return pltpu.make_async_copy(
            ot_v.at[k],
            out_hbm.at[s, pl.ds(0, HIDDEN), pl.ds(b0, BW)],
            osem[k],
        )

    for k in range(GBUF):
        g_cp(k, k).start()

    def outer(T, carry):
        for k in range(GBUF):
            s = T * GBUF + k
            g_cp(s, k).wait()

            @pl.when(T > 0)
            def _():
                o_cp(s - GBUF, k).wait()

            def group(g, gc):
                for c in range(HIDDEN // LANES):
                    col = c * LANES + lane_iota
                    for j in range(LANES):
                        r = g * LANES + j
                        seg = unit_v[k, r, pl.ds(c * LANES, LANES)]
                        plsc.store_scatter(
                            ot_v.at[k], [col, jnp.full((LANES,), r, jnp.int32)], seg
                        )
                return gc

            lax.fori_loop(0, BW // LANES, group, 0)

            def zfix(g, gc):
                iv = idx_v[s, pl.ds(g * LANES, LANES)]
                nz = plsc.all_reduce_population_count(iv == 0)

                @pl.when(nz[0] > 0)
                def _():
                    for j in range(LANES):
                        @pl.when(iv[j] == 0)
                        def _():
                            r = g * LANES + j
                            for c in range(HIDDEN // LANES):
                                plsc.store_scatter(
                                    ot_v.at[k],
                                    [
                                        c * LANES + lane_iota,
                                        jnp.full((LANES,), r, jnp.int32),
                                    ],
                                    zeros,
                                )
                return gc

            lax.fori_loop(0, BW // LANES, zfix, 0)
            o_cp(s, k).start()

            @pl.when(s + GBUF < SEQ)
            def _():
                g_cp(s + GBUF, k).start()

        return carry

    lax.fori_loop(0, GOUT, outer, 0)
    for k in range(GBUF):
        o_cp(SEQ - GBUF + k, k).wait()


def kernel(x, shared_weights):
    xt = x.astype(jnp.int32).T
    wt = shared_weights.T
    tab = _repack(wt)
    out_st = _embed(xt, tab)
    return jnp.transpose(out_st, (2, 0, 1))
